# baseline (speedup 1.0000x reference)
import functools

import jax
import jax.numpy as jnp
from jax import lax
from jax.experimental import pallas as pl
from jax.experimental.pallas import tpu as pltpu
from jax.experimental.pallas import tpu_sc as plsc

_KNN_ROWS = 256
_TP = 128


def _mm(a, b):
    return lax.dot_general(a, b, (((1,), (0,)), ((), ())),
                           preferred_element_type=jnp.float32)


def _knn_body(xt_ref, xa_ref, ptsT_ref, wqkv_ref, wp1_ref,
              idx_ref, q_ref, u_ref, kvu_ref, *, n, rows, k, dim, ph):
    b = pl.program_id(0)
    xt = xt_ref[0]
    xa = xa_ref[0]

    qkv_t = lax.dot_general(ptsT_ref[0], wqkv_ref[...], (((0,), (1,)), ((), ())),
                            preferred_element_type=jnp.float32)

    u = _mm(xt, wp1_ref[...])
    q_ref[0] = qkv_t[:, :dim]
    u_ref[0] = u
    kv = _pack2(qkv_t[:, dim:2 * dim], qkv_t[:, 2 * dim:])
    up = _pack2(u[:, :ph // 2], u[:, ph // 2:])
    kvu_ref[0] = jnp.concatenate(
        [kv, up, jnp.zeros((rows, dim - ph // 2), jnp.int32)], axis=1)

    dot = lax.dot_general(xt, xa, (((1,), (0,)), ((), ())),
                          preferred_element_type=jnp.float32)
    sq_rows = jnp.sum(xt * xt, axis=1, keepdims=True)
    sq_all = jnp.sum(xa * xa, axis=0, keepdims=True)
    d = sq_rows + sq_all - 2.0 * dot

    iota_n = lax.broadcasted_iota(jnp.int32, (rows, n), 1)
    iota_k = lax.broadcasted_iota(jnp.int32, (rows, k), 1)
    base = b * n
    idx_acc = jnp.zeros((rows, k), jnp.int32)
    for t in range(k):
        m = jnp.min(d, axis=1, keepdims=True)
        cand = jnp.where(d == m, iota_n, n)
        ai = jnp.min(cand, axis=1, keepdims=True)
        idx_acc = jnp.where(iota_k == t, ai + base, idx_acc)
        d = jnp.where(iota_n == ai, jnp.float32(jnp.inf), d)
    idx_ref[0] = idx_acc


def _pack2(a, b):
    ai = lax.bitcast_convert_type(a, jnp.uint32)
    bi = lax.bitcast_convert_type(b, jnp.uint32)
    one = jnp.uint32(1)
    rnd = jnp.uint32(0x7FFF)
    ar = (ai + rnd + ((ai >> 16) & one)) & jnp.uint32(0xFFFF0000)
    br = (bi + rnd + ((bi >> 16) & one)) >> 16
    return lax.bitcast_convert_type(ar | br, jnp.int32)


def _sc_gather(kvu_tab, idx_flat):
    tok = idx_flat.shape[0]
    width = kvu_tab.shape[1]
    wout = kvu_tab.shape[1]
    info = plsc.get_sparse_core_info()
    nc, ns = info.num_cores, info.num_subcores
    nw = nc * ns
    rpw = tok // nw
    ch = 128
    nb = 2
    nch = rpw // ch

    @functools.partial(
        pl.kernel,
        mesh=plsc.VectorSubcoreMesh(core_axis_name="c", subcore_axis_name="s"),
        out_type=jax.ShapeDtypeStruct((tok, wout), jnp.int32),
        scratch_types=[pltpu.VMEM((nb, ch), jnp.int32),
                       pltpu.VMEM((nb, ch, width), jnp.int32),
                       pltpu.SemaphoreType.DMA],
    )
    def gather(kvu_h, idx_h, out_h, idxv, bufs, sem):
        wid = lax.axis_index("s") * nc + lax.axis_index("c")
        base = wid * rpw

        def body(g, carry):
            c0 = g * nb
            for j in range(nb):
                pltpu.sync_copy(idx_h.at[pl.ds(base + (c0 + j) * ch, ch)],
                                idxv.at[j])
            handles = [pltpu.async_copy(kvu_h.at[idxv.at[j]], bufs.at[j], sem)
                       for j in range(nb)]
            for j in range(nb):
                handles[j].wait()
                pltpu.sync_copy(bufs.at[j],
                                out_h.at[pl.ds(base + (c0 + j) * ch, ch)])
            return carry

        lax.fori_loop(0, nch // nb, body, 0)

    return gather(kvu_tab, idx_flat)


def _fused_body(kvu_ref, q_ref, uc_ref,
                bp1_ref, wp2_ref, bp2_ref,
                wa1_ref, ba1_ref, wa2_ref, ba2_ref, out_ref, *,
                tp, k, dim, ph):
    t = tp * k
    xi = lax.bitcast_convert_type(kvu_ref[...], jnp.uint32)
    hi = lax.bitcast_convert_type(xi & jnp.uint32(0xFFFF0000), jnp.float32)
    lo = lax.bitcast_convert_type(xi << 16, jnp.float32)
    kg = hi[:, :dim]
    vg = lo[:, :dim]
    ug = jnp.concatenate(
        [hi[:, dim:dim + ph // 2], lo[:, dim:dim + ph // 2]], axis=1)
    q = q_ref[...]
    uc = uc_ref[...]

    ucr = jnp.broadcast_to(uc[:, None, :], (tp, k, ph)).reshape(t, ph)
    h = jnp.maximum(ucr - ug + bp1_ref[...], 0.0)
    rpe = _mm(h.astype(jnp.bfloat16), wp2_ref[...]) + bp2_ref[...]

    qr = jnp.broadcast_to(q[:, None, :], (tp, k, dim)).reshape(t, dim)
    pre = qr - kg + rpe
    a = jnp.maximum(_mm(pre.astype(jnp.bfloat16), wa1_ref[...]) + ba1_ref[...],
                    0.0)
    sim = _mm(a.astype(jnp.bfloat16), wa2_ref[...]) + ba2_ref[...]

    s3 = sim.reshape(tp, k, dim)
    mx = jnp.max(s3, axis=1, keepdims=True)
    e = jnp.exp(s3 - mx)
    den = jnp.sum(e, axis=1, keepdims=True)
    attn = e / den

    vv = (vg + rpe).reshape(tp, k, dim)
    out_ref[0] = jnp.transpose(jnp.sum(attn * vv, axis=1))


def kernel(xyz, points, W_qkv, Wp1, bp1, Wp2, bp2, Wa1, ba1, Wa2, ba2):
    b, _, n = xyz.shape
    dim = points.shape[1]
    ph = Wp1.shape[0]
    hid = Wa1.shape[0]
    k = 16

    xyz_t16 = jnp.pad(jnp.transpose(xyz, (0, 2, 1)), ((0, 0), (0, 0), (0, 13)))
    xyz_p16 = jnp.pad(xyz, ((0, 0), (0, 13), (0, 0)))
    wp1t = jnp.pad(Wp1.T, ((0, 13), (0, 0)))
    wp2t_bf = Wp2.T.astype(jnp.bfloat16)
    wa1t_bf = Wa1.T.astype(jnp.bfloat16)
    wa2t_bf = Wa2.T.astype(jnp.bfloat16)
    bp1r = bp1.reshape(1, ph)
    bp2r = bp2.reshape(1, dim)
    ba1r = ba1.reshape(1, hid)
    ba2r = ba2.reshape(1, dim)

    rows = _KNN_ROWS
    width = 2 * dim

    def _half(xyz_t16, xyz_p16, points, b):
        idx, q_t, u_t, kvu_t = pl.pallas_call(
        functools.partial(_knn_body, n=n, rows=rows, k=k, dim=dim, ph=ph),
        grid=(b, n // rows),
        in_specs=[pl.BlockSpec((1, rows, 16), lambda bi, ri: (bi, ri, 0)),
                  pl.BlockSpec((1, 16, n), lambda bi, ri: (bi, 0, 0)),
                  pl.BlockSpec((1, dim, rows), lambda bi, ri: (bi, 0, ri)),
                  pl.BlockSpec((3 * dim, dim), lambda bi, ri: (0, 0)),
                  pl.BlockSpec((16, ph), lambda bi, ri: (0, 0))],
        out_specs=[pl.BlockSpec((1, rows, k), lambda bi, ri: (bi, ri, 0)),
                   pl.BlockSpec((1, rows, dim), lambda bi, ri: (bi, ri, 0)),
                   pl.BlockSpec((1, rows, ph), lambda bi, ri: (bi, ri, 0)),
                   pl.BlockSpec((1, rows, width), lambda bi, ri: (bi, ri, 0))],
        out_shape=[jax.ShapeDtypeStruct((b, n, k), jnp.int32),
                   jax.ShapeDtypeStruct((b, n, dim), jnp.float32),
                   jax.ShapeDtypeStruct((b, n, ph), jnp.float32),
                   jax.ShapeDtypeStruct((b, n, width), jnp.int32)],
        )(xyz_t16, xyz_p16, points, W_qkv, wp1t)

        kvu_g = _sc_gather(kvu_t.reshape(b * n, width),
                           idx.reshape(b * n * k))

        tp = _TP
        tb = tp * k
        nbp = n // tp
        return pl.pallas_call(
        functools.partial(_fused_body, tp=tp, k=k, dim=dim, ph=ph),
        grid=((b * n) // tp,),
        in_specs=[
            pl.BlockSpec((tb, width), lambda i: (i, 0)),
            pl.BlockSpec((tp, dim), lambda i: (i, 0)),
            pl.BlockSpec((tp, ph), lambda i: (i, 0)),
            pl.BlockSpec((1, ph), lambda i: (0, 0)),
            pl.BlockSpec((ph, dim), lambda i: (0, 0)),
            pl.BlockSpec((1, dim), lambda i: (0, 0)),
            pl.BlockSpec((dim, hid), lambda i: (0, 0)),
            pl.BlockSpec((1, hid), lambda i: (0, 0)),
            pl.BlockSpec((hid, dim), lambda i: (0, 0)),
            pl.BlockSpec((1, dim), lambda i: (0, 0)),
        ],
        out_specs=pl.BlockSpec((1, dim, tp),
                               lambda i: (i // nbp, 0, i % nbp)),
        out_shape=jax.ShapeDtypeStruct((b, dim, n), jnp.float32),
        )(kvu_g, q_t.reshape(b * n, dim), u_t.reshape(b * n, ph),
          bp1r, wp2t_bf, bp2r, wa1t_bf, ba1r, wa2t_bf, ba2r)

    halves = [_half(xyz_t16[i:i + 1], xyz_p16[i:i + 1],
                    points[i:i + 1], 1) for i in range(b)]
    return jnp.concatenate(halves, axis=0)


# speedup vs baseline: 1.1909x; 1.1909x over previous
import functools

import jax
import jax.numpy as jnp
from jax import lax
from jax.experimental import pallas as pl
from jax.experimental.pallas import tpu as pltpu
from jax.experimental.pallas import tpu_sc as plsc

_KNN_ROWS = 256
_TP = 128


def _mm(a, b):
    return lax.dot_general(a, b, (((1,), (0,)), ((), ())),
                           preferred_element_type=jnp.float32)


def _knn_body(xt_ref, xa_ref, ptsT_ref, wqkv_ref, wp1_ref,
              idx_ref, q_ref, u_ref, kvu_ref, *, n, rows, k, dim, ph):
    b = pl.program_id(0)
    xt = xt_ref[0]
    xa = xa_ref[0]

    qkv_t = lax.dot_general(ptsT_ref[0], wqkv_ref[...], (((0,), (1,)), ((), ())),
                            preferred_element_type=jnp.float32)

    u = _mm(xt, wp1_ref[...])
    q_ref[0] = qkv_t[:, :dim]
    u_ref[0] = u
    kv = _pack2(qkv_t[:, dim:2 * dim], qkv_t[:, 2 * dim:])
    up = _pack2(u[:, :ph // 2], u[:, ph // 2:])
    kvu_ref[0] = jnp.concatenate(
        [kv, up, jnp.zeros((rows, dim - ph // 2), jnp.int32)], axis=1)

    dot = lax.dot_general(xt, xa, (((1,), (0,)), ((), ())),
                          preferred_element_type=jnp.float32)
    sq_rows = jnp.sum(xt * xt, axis=1, keepdims=True)
    sq_all = jnp.sum(xa * xa, axis=0, keepdims=True)
    d = sq_rows + sq_all - 2.0 * dot

    iota_nf = lax.broadcasted_iota(jnp.int32, (rows, n), 1).astype(jnp.float32)
    iota_k = lax.broadcasted_iota(jnp.int32, (rows, k), 1)
    nf = jnp.float32(n)
    base = b * n
    idx_acc = jnp.zeros((rows, k), jnp.int32)
    for t in range(k):
        m = jnp.min(d, axis=1, keepdims=True)
        cand = jnp.where(d == m, iota_nf, nf)
        ai = jnp.min(cand, axis=1, keepdims=True)
        idx_acc = jnp.where(iota_k == t, ai.astype(jnp.int32) + base, idx_acc)
        d = jnp.where(iota_nf == ai, jnp.float32(jnp.inf), d)
    idx_ref[0] = idx_acc


def _pack2(a, b):
    ai = lax.bitcast_convert_type(a, jnp.uint32)
    bi = lax.bitcast_convert_type(b, jnp.uint32)
    one = jnp.uint32(1)
    rnd = jnp.uint32(0x7FFF)
    ar = (ai + rnd + ((ai >> 16) & one)) & jnp.uint32(0xFFFF0000)
    br = (bi + rnd + ((bi >> 16) & one)) >> 16
    return lax.bitcast_convert_type(ar | br, jnp.int32)


def _sc_gather(kvu_tab, idx_flat):
    tok = idx_flat.shape[0]
    width = kvu_tab.shape[1]
    wout = kvu_tab.shape[1]
    info = plsc.get_sparse_core_info()
    nc, ns = info.num_cores, info.num_subcores
    nw = nc * ns
    rpw = tok // nw
    ch = 128
    nb = 2
    nch = rpw // ch

    @functools.partial(
        pl.kernel,
        mesh=plsc.VectorSubcoreMesh(core_axis_name="c", subcore_axis_name="s"),
        out_type=jax.ShapeDtypeStruct((tok, wout), jnp.int32),
        scratch_types=[pltpu.VMEM((nb, ch), jnp.int32),
                       pltpu.VMEM((nb, ch, width), jnp.int32),
                       pltpu.SemaphoreType.DMA],
    )
    def gather(kvu_h, idx_h, out_h, idxv, bufs, sem):
        wid = lax.axis_index("s") * nc + lax.axis_index("c")
        base = wid * rpw

        def body(g, carry):
            c0 = g * nb
            for j in range(nb):
                pltpu.sync_copy(idx_h.at[pl.ds(base + (c0 + j) * ch, ch)],
                                idxv.at[j])
            handles = [pltpu.async_copy(kvu_h.at[idxv.at[j]], bufs.at[j], sem)
                       for j in range(nb)]
            for j in range(nb):
                handles[j].wait()
                pltpu.sync_copy(bufs.at[j],
                                out_h.at[pl.ds(base + (c0 + j) * ch, ch)])
            return carry

        lax.fori_loop(0, nch // nb, body, 0)

    return gather(kvu_tab, idx_flat)


def _fused_body(kvu_ref, q_ref, uc_ref,
                bp1_ref, wp2_ref, bp2_ref,
                wa1_ref, ba1_ref, wa2_ref, ba2_ref, out_ref, *,
                tp, k, dim, ph):
    t = tp * k
    xi = lax.bitcast_convert_type(kvu_ref[...], jnp.uint32)
    hi = lax.bitcast_convert_type(xi & jnp.uint32(0xFFFF0000), jnp.float32)
    lo = lax.bitcast_convert_type(xi << 16, jnp.float32)
    kg = hi[:, :dim]
    vg = lo[:, :dim]
    ug = jnp.concatenate(
        [hi[:, dim:dim + ph // 2], lo[:, dim:dim + ph // 2]], axis=1)
    q = q_ref[...]
    uc = uc_ref[...]

    ucr = jnp.broadcast_to(uc[:, None, :], (tp, k, ph)).reshape(t, ph)
    h = jnp.maximum(ucr - ug + bp1_ref[...], 0.0)
    rpe = _mm(h.astype(jnp.bfloat16), wp2_ref[...]) + bp2_ref[...]

    qr = jnp.broadcast_to(q[:, None, :], (tp, k, dim)).reshape(t, dim)
    pre = qr - kg + rpe
    a = jnp.maximum(_mm(pre.astype(jnp.bfloat16), wa1_ref[...]) + ba1_ref[...],
                    0.0)
    sim = _mm(a.astype(jnp.bfloat16), wa2_ref[...]) + ba2_ref[...]

    s3 = sim.reshape(tp, k, dim)
    mx = jnp.max(s3, axis=1, keepdims=True)
    e = jnp.exp(s3 - mx)
    den = jnp.sum(e, axis=1, keepdims=True)
    attn = e / den

    vv = (vg + rpe).reshape(tp, k, dim)
    out_ref[0] = jnp.transpose(jnp.sum(attn * vv, axis=1))


def kernel(xyz, points, W_qkv, Wp1, bp1, Wp2, bp2, Wa1, ba1, Wa2, ba2):
    b, _, n = xyz.shape
    dim = points.shape[1]
    ph = Wp1.shape[0]
    hid = Wa1.shape[0]
    k = 16

    xyz_t16 = jnp.pad(jnp.transpose(xyz, (0, 2, 1)), ((0, 0), (0, 0), (0, 13)))
    xyz_p16 = jnp.pad(xyz, ((0, 0), (0, 13), (0, 0)))
    wp1t = jnp.pad(Wp1.T, ((0, 13), (0, 0)))
    wp2t_bf = Wp2.T.astype(jnp.bfloat16)
    wa1t_bf = Wa1.T.astype(jnp.bfloat16)
    wa2t_bf = Wa2.T.astype(jnp.bfloat16)
    bp1r = bp1.reshape(1, ph)
    bp2r = bp2.reshape(1, dim)
    ba1r = ba1.reshape(1, hid)
    ba2r = ba2.reshape(1, dim)

    rows = _KNN_ROWS
    width = 2 * dim

    def _half(xyz_t16, xyz_p16, points, b):
        idx, q_t, u_t, kvu_t = pl.pallas_call(
        functools.partial(_knn_body, n=n, rows=rows, k=k, dim=dim, ph=ph),
        grid=(b, n // rows),
        in_specs=[pl.BlockSpec((1, rows, 16), lambda bi, ri: (bi, ri, 0)),
                  pl.BlockSpec((1, 16, n), lambda bi, ri: (bi, 0, 0)),
                  pl.BlockSpec((1, dim, rows), lambda bi, ri: (bi, 0, ri)),
                  pl.BlockSpec((3 * dim, dim), lambda bi, ri: (0, 0)),
                  pl.BlockSpec((16, ph), lambda bi, ri: (0, 0))],
        out_specs=[pl.BlockSpec((1, rows, k), lambda bi, ri: (bi, ri, 0)),
                   pl.BlockSpec((1, rows, dim), lambda bi, ri: (bi, ri, 0)),
                   pl.BlockSpec((1, rows, ph), lambda bi, ri: (bi, ri, 0)),
                   pl.BlockSpec((1, rows, width), lambda bi, ri: (bi, ri, 0))],
        out_shape=[jax.ShapeDtypeStruct((b, n, k), jnp.int32),
                   jax.ShapeDtypeStruct((b, n, dim), jnp.float32),
                   jax.ShapeDtypeStruct((b, n, ph), jnp.float32),
                   jax.ShapeDtypeStruct((b, n, width), jnp.int32)],
        )(xyz_t16, xyz_p16, points, W_qkv, wp1t)

        kvu_g = _sc_gather(kvu_t.reshape(b * n, width),
                           idx.reshape(b * n * k))

        tp = _TP
        tb = tp * k
        nbp = n // tp
        return pl.pallas_call(
        functools.partial(_fused_body, tp=tp, k=k, dim=dim, ph=ph),
        grid=((b * n) // tp,),
        in_specs=[
            pl.BlockSpec((tb, width), lambda i: (i, 0)),
            pl.BlockSpec((tp, dim), lambda i: (i, 0)),
            pl.BlockSpec((tp, ph), lambda i: (i, 0)),
            pl.BlockSpec((1, ph), lambda i: (0, 0)),
            pl.BlockSpec((ph, dim), lambda i: (0, 0)),
            pl.BlockSpec((1, dim), lambda i: (0, 0)),
            pl.BlockSpec((dim, hid), lambda i: (0, 0)),
            pl.BlockSpec((1, hid), lambda i: (0, 0)),
            pl.BlockSpec((hid, dim), lambda i: (0, 0)),
            pl.BlockSpec((1, dim), lambda i: (0, 0)),
        ],
        out_specs=pl.BlockSpec((1, dim, tp),
                               lambda i: (i // nbp, 0, i % nbp)),
        out_shape=jax.ShapeDtypeStruct((b, dim, n), jnp.float32),
        )(kvu_g, q_t.reshape(b * n, dim), u_t.reshape(b * n, ph),
          bp1r, wp2t_bf, bp2r, wa1t_bf, ba1r, wa2t_bf, ba2r)

    bh = b // 2
    halves = [_half(xyz_t16[i * bh:(i + 1) * bh], xyz_p16[i * bh:(i + 1) * bh],
                    points[i * bh:(i + 1) * bh], bh) for i in range(2)]
    return jnp.concatenate(halves, axis=0)


# speedup vs baseline: 1.1993x; 1.0070x over previous
import functools

import jax
import jax.numpy as jnp
from jax import lax
from jax.experimental import pallas as pl
from jax.experimental.pallas import tpu as pltpu
from jax.experimental.pallas import tpu_sc as plsc

_KNN_ROWS = 256
_TP = 128


def _mm(a, b):
    return lax.dot_general(a, b, (((1,), (0,)), ((), ())),
                           preferred_element_type=jnp.float32)


def _knn_body(xt_ref, xa_ref, ptsT_ref, wqkv_ref, wp1_ref,
              idx_ref, q_ref, u_ref, kvu_ref, *, n, rows, k, dim, ph):
    b = pl.program_id(0)
    xt = xt_ref[0]
    xa = xa_ref[0]

    qkv_t = lax.dot_general(ptsT_ref[0], wqkv_ref[...], (((0,), (1,)), ((), ())),
                            preferred_element_type=jnp.float32)

    u = _mm(xt, wp1_ref[...])
    q_ref[0] = qkv_t[:, :dim]
    u_ref[0] = u
    kv = _pack2(qkv_t[:, dim:2 * dim], qkv_t[:, 2 * dim:])
    up = _pack2(u[:, :ph // 2], u[:, ph // 2:])
    kvu_ref[0] = jnp.concatenate(
        [kv, up, jnp.zeros((rows, dim - ph // 2), jnp.int32)], axis=1)

    dot = lax.dot_general(xt, xa, (((1,), (0,)), ((), ())),
                          preferred_element_type=jnp.float32)
    sq_rows = jnp.sum(xt * xt, axis=1, keepdims=True)
    sq_all = jnp.sum(xa * xa, axis=0, keepdims=True)
    d = sq_rows + sq_all - 2.0 * dot

    iota_nf = lax.broadcasted_iota(jnp.int32, (rows, n), 1).astype(jnp.float32)
    iota_k = lax.broadcasted_iota(jnp.int32, (rows, k), 1)
    nf = jnp.float32(n)
    base = b * n
    idx_acc = jnp.zeros((rows, k), jnp.int32)
    for t in range(k):
        m = jnp.min(d, axis=1, keepdims=True)
        cand = jnp.where(d == m, iota_nf, nf)
        ai = jnp.min(cand, axis=1, keepdims=True)
        idx_acc = jnp.where(iota_k == t, ai.astype(jnp.int32) + base, idx_acc)
        if t < k - 1:
            d = jnp.where(iota_nf == ai, jnp.float32(jnp.inf), d)
    idx_ref[0] = idx_acc


def _pack2(a, b):
    ai = lax.bitcast_convert_type(a, jnp.uint32)
    bi = lax.bitcast_convert_type(b, jnp.uint32)
    one = jnp.uint32(1)
    rnd = jnp.uint32(0x7FFF)
    ar = (ai + rnd + ((ai >> 16) & one)) & jnp.uint32(0xFFFF0000)
    br = (bi + rnd + ((bi >> 16) & one)) >> 16
    return lax.bitcast_convert_type(ar | br, jnp.int32)


def _sc_gather(kvu_tab, idx_flat):
    tok = idx_flat.shape[0]
    width = kvu_tab.shape[1]
    wout = kvu_tab.shape[1]
    info = plsc.get_sparse_core_info()
    nc, ns = info.num_cores, info.num_subcores
    nw = nc * ns
    rpw = tok // nw
    ch = 128
    nb = 3
    nch = rpw // ch

    @functools.partial(
        pl.kernel,
        mesh=plsc.VectorSubcoreMesh(core_axis_name="c", subcore_axis_name="s"),
        out_type=jax.ShapeDtypeStruct((tok, wout), jnp.int32),
        scratch_types=[pltpu.VMEM((nb, ch), jnp.int32),
                       pltpu.VMEM((nb, ch, width), jnp.int32),
                       pltpu.SemaphoreType.DMA],
    )
    def gather(kvu_h, idx_h, out_h, idxv, bufs, sem):
        wid = lax.axis_index("s") * nc + lax.axis_index("c")
        base = wid * rpw

        def run(c0, cnt):
            for j in range(cnt):
                pltpu.sync_copy(idx_h.at[pl.ds(base + (c0 + j) * ch, ch)],
                                idxv.at[j])
            handles = [pltpu.async_copy(kvu_h.at[idxv.at[j]], bufs.at[j], sem)
                       for j in range(cnt)]
            for j in range(cnt):
                handles[j].wait()
                pltpu.sync_copy(bufs.at[j],
                                out_h.at[pl.ds(base + (c0 + j) * ch, ch)])

        def body(g, carry):
            run(g * nb, nb)
            return carry

        lax.fori_loop(0, nch // nb, body, 0)
        if nch % nb:
            run((nch // nb) * nb, nch % nb)

    return gather(kvu_tab, idx_flat)


def _fused_body(kvu_ref, q_ref, uc_ref,
                bp1_ref, wp2_ref, bp2_ref,
                wa1_ref, ba1_ref, wa2_ref, ba2_ref, out_ref, *,
                tp, k, dim, ph):
    t = tp * k
    xi = lax.bitcast_convert_type(kvu_ref[...], jnp.uint32)
    hi = lax.bitcast_convert_type(xi & jnp.uint32(0xFFFF0000), jnp.float32)
    lo = lax.bitcast_convert_type(xi << 16, jnp.float32)
    kg = hi[:, :dim]
    vg = lo[:, :dim]
    ug = jnp.concatenate(
        [hi[:, dim:dim + ph // 2], lo[:, dim:dim + ph // 2]], axis=1)
    q = q_ref[...]
    uc = uc_ref[...]

    ucr = jnp.broadcast_to(uc[:, None, :], (tp, k, ph)).reshape(t, ph)
    h = jnp.maximum(ucr - ug + bp1_ref[...], 0.0)
    rpe = _mm(h.astype(jnp.bfloat16), wp2_ref[...]) + bp2_ref[...]

    qr = jnp.broadcast_to(q[:, None, :], (tp, k, dim)).reshape(t, dim)
    pre = qr - kg + rpe
    a = jnp.maximum(_mm(pre.astype(jnp.bfloat16), wa1_ref[...]) + ba1_ref[...],
                    0.0)
    sim = _mm(a.astype(jnp.bfloat16), wa2_ref[...]) + ba2_ref[...]

    s3 = sim.reshape(tp, k, dim)
    mx = jnp.max(s3, axis=1, keepdims=True)
    e = jnp.exp(s3 - mx)
    den = jnp.sum(e, axis=1, keepdims=True)
    attn = e / den

    vv = (vg + rpe).reshape(tp, k, dim)
    out_ref[0] = jnp.transpose(jnp.sum(attn * vv, axis=1))


def kernel(xyz, points, W_qkv, Wp1, bp1, Wp2, bp2, Wa1, ba1, Wa2, ba2):
    b, _, n = xyz.shape
    dim = points.shape[1]
    ph = Wp1.shape[0]
    hid = Wa1.shape[0]
    k = 16

    xyz_t16 = jnp.pad(jnp.transpose(xyz, (0, 2, 1)), ((0, 0), (0, 0), (0, 13)))
    xyz_p16 = jnp.pad(xyz, ((0, 0), (0, 13), (0, 0)))
    wp1t = jnp.pad(Wp1.T, ((0, 13), (0, 0)))
    wp2t_bf = Wp2.T.astype(jnp.bfloat16)
    wa1t_bf = Wa1.T.astype(jnp.bfloat16)
    wa2t_bf = Wa2.T.astype(jnp.bfloat16)
    bp1r = bp1.reshape(1, ph)
    bp2r = bp2.reshape(1, dim)
    ba1r = ba1.reshape(1, hid)
    ba2r = ba2.reshape(1, dim)

    rows = _KNN_ROWS
    width = 2 * dim

    def _half(xyz_t16, xyz_p16, points, b):
        idx, q_t, u_t, kvu_t = pl.pallas_call(
        functools.partial(_knn_body, n=n, rows=rows, k=k, dim=dim, ph=ph),
        grid=(b, n // rows),
        in_specs=[pl.BlockSpec((1, rows, 16), lambda bi, ri: (bi, ri, 0)),
                  pl.BlockSpec((1, 16, n), lambda bi, ri: (bi, 0, 0)),
                  pl.BlockSpec((1, dim, rows), lambda bi, ri: (bi, 0, ri)),
                  pl.BlockSpec((3 * dim, dim), lambda bi, ri: (0, 0)),
                  pl.BlockSpec((16, ph), lambda bi, ri: (0, 0))],
        out_specs=[pl.BlockSpec((1, rows, k), lambda bi, ri: (bi, ri, 0)),
                   pl.BlockSpec((1, rows, dim), lambda bi, ri: (bi, ri, 0)),
                   pl.BlockSpec((1, rows, ph), lambda bi, ri: (bi, ri, 0)),
                   pl.BlockSpec((1, rows, width), lambda bi, ri: (bi, ri, 0))],
        out_shape=[jax.ShapeDtypeStruct((b, n, k), jnp.int32),
                   jax.ShapeDtypeStruct((b, n, dim), jnp.float32),
                   jax.ShapeDtypeStruct((b, n, ph), jnp.float32),
                   jax.ShapeDtypeStruct((b, n, width), jnp.int32)],
        )(xyz_t16, xyz_p16, points, W_qkv, wp1t)

        kvu_g = _sc_gather(kvu_t.reshape(b * n, width),
                           idx.reshape(b * n * k))

        tp = _TP
        tb = tp * k
        nbp = n // tp
        return pl.pallas_call(
        functools.partial(_fused_body, tp=tp, k=k, dim=dim, ph=ph),
        grid=((b * n) // tp,),
        in_specs=[
            pl.BlockSpec((tb, width), lambda i: (i, 0)),
            pl.BlockSpec((tp, dim), lambda i: (i, 0)),
            pl.BlockSpec((tp, ph), lambda i: (i, 0)),
            pl.BlockSpec((1, ph), lambda i: (0, 0)),
            pl.BlockSpec((ph, dim), lambda i: (0, 0)),
            pl.BlockSpec((1, dim), lambda i: (0, 0)),
            pl.BlockSpec((dim, hid), lambda i: (0, 0)),
            pl.BlockSpec((1, hid), lambda i: (0, 0)),
            pl.BlockSpec((hid, dim), lambda i: (0, 0)),
            pl.BlockSpec((1, dim), lambda i: (0, 0)),
        ],
        out_specs=pl.BlockSpec((1, dim, tp),
                               lambda i: (i // nbp, 0, i % nbp)),
        out_shape=jax.ShapeDtypeStruct((b, dim, n), jnp.float32),
        )(kvu_g, q_t.reshape(b * n, dim), u_t.reshape(b * n, ph),
          bp1r, wp2t_bf, bp2r, wa1t_bf, ba1r, wa2t_bf, ba2r)

    bh = b // 2
    halves = [_half(xyz_t16[i * bh:(i + 1) * bh], xyz_p16[i * bh:(i + 1) * bh],
                    points[i * bh:(i + 1) * bh], bh) for i in range(2)]
    return jnp.concatenate(halves, axis=0)
